# Initial kernel scaffold; baseline (speedup 1.0000x reference)
import jax
import jax.numpy as jnp
from jax.experimental import pallas as pl


def kernel(x, video_names, shape, qkv_w, proj_w, proj_b):
    raise NotImplementedError("write your pallas kernel here")


# speedup vs baseline: 1.1272x; 1.1272x over previous
import functools

import jax
import jax.numpy as jnp
import numpy as np
from jax.experimental import pallas as pl

B, N, C = 2, 2048, 1024
H = 16
HD = C // H
BQ = 512


def _mha_kernel(x_ref, w_ref, pw_ref, pb_ref, o_ref):
    h = pl.program_id(1)
    xb = x_ref[0]
    wq = w_ref[0, 0]
    wk = w_ref[1, 0]
    wv = w_ref[2, 0]
    wp = pw_ref[0]

    dot = functools.partial(
        jax.lax.dot_general, preferred_element_type=jnp.float32
    )
    q = dot(xb, wq, (((1,), (1,)), ((), ())))
    k = dot(xb, wk, (((1,), (1,)), ((), ())))
    v = dot(xb, wv, (((1,), (1,)), ((), ())))

    scale = 1.0 / np.sqrt(HD)

    @pl.when(h == 0)
    def _init():
        o_ref[0] = jnp.broadcast_to(pb_ref[0], (N, C))

    for i in range(N // BQ):
        qi = q[i * BQ:(i + 1) * BQ]
        s = dot(qi, k, (((1,), (1,)), ((), ()))) * scale
        m = jnp.max(s, axis=-1, keepdims=True)
        p = jnp.exp(s - m)
        l = jnp.sum(p, axis=-1, keepdims=True)
        o = dot(p, v, (((1,), (0,)), ((), ()))) / l
        contrib = dot(o, wp, (((1,), (0,)), ((), ())))
        o_ref[0, pl.ds(i * BQ, BQ), :] += contrib


def kernel(x, video_names, shape, qkv_w, proj_w, proj_b):
    del video_names, shape
    w = qkv_w.reshape(3, H, HD, C)
    pw = jnp.transpose(proj_w.reshape(C, H, HD), (1, 2, 0))
    pb = proj_b.reshape(1, C)

    out = pl.pallas_call(
        _mha_kernel,
        grid=(B, H),
        in_specs=[
            pl.BlockSpec((1, N, C), lambda b, h: (b, 0, 0)),
            pl.BlockSpec((3, 1, HD, C), lambda b, h: (0, h, 0, 0)),
            pl.BlockSpec((1, HD, C), lambda b, h: (h, 0, 0)),
            pl.BlockSpec((1, C), lambda b, h: (0, 0)),
        ],
        out_specs=pl.BlockSpec((1, N, C), lambda b, h: (b, 0, 0)),
        out_shape=jax.ShapeDtypeStruct((B, N, C), jnp.float32),
    )(x, w, pw, pb)
    return out
